# baseline (speedup 1.0000x reference)
import functools

import jax
import jax.numpy as jnp
from jax.experimental import pallas as pl
from jax.experimental.pallas import tpu as pltpu
from jax.experimental.pallas import tpu_sc as plsc

NPOINT_ = 1024
K_ = 32
N_ = 4096
QB_ = 512
CPAD_ = 128


def _fps_body(xs_ref, ys_ref, zs_ref, o_ref):
    x = xs_ref[0]
    y = ys_ref[0]
    z = zs_ref[0]
    R = x.shape[0]
    piota = jax.lax.broadcasted_iota(jnp.int32, (R, NPOINT_), 1)

    def step(i, carry):
        dist, cx, cy, cz, ax, ay, az = carry
        hit = piota == i
        ax = jnp.where(hit, cx, ax)
        ay = jnp.where(hit, cy, ay)
        az = jnp.where(hit, cz, az)
        dx = x - cx
        dy = y - cy
        dz = z - cz
        d = (dx * dx + dy * dy) + dz * dz
        dist = jnp.minimum(dist, d)
        v, tx, ty, tz = dist, x, y, z
        size = N_ // 2
        while size >= 1:
            take = v[:, size:2 * size] > v[:, :size]
            v = jnp.where(take, v[:, size:2 * size], v[:, :size])
            tx = jnp.where(take, tx[:, size:2 * size], tx[:, :size])
            ty = jnp.where(take, ty[:, size:2 * size], ty[:, :size])
            tz = jnp.where(take, tz[:, size:2 * size], tz[:, :size])
            size //= 2
        return dist, tx, ty, tz, ax, ay, az

    dist0 = jnp.full((R, N_), 1e10, jnp.float32)
    acc0 = jnp.zeros((R, NPOINT_), jnp.float32)
    _, _, _, _, ax, ay, az = jax.lax.fori_loop(
        0, NPOINT_, step,
        (dist0, x[:, 0:1], y[:, 0:1], z[:, 0:1], acc0, acc0, acc0))
    o_ref[0, 0] = ax
    o_ref[1, 0] = ay
    o_ref[2, 0] = az


def _fps_points(xyz_flat, interpret=False):
    xs = xyz_flat[:, 0, :].reshape(2, 3, N_)
    ys = xyz_flat[:, 1, :].reshape(2, 3, N_)
    zs = xyz_flat[:, 2, :].reshape(2, 3, N_)
    return pl.pallas_call(
        _fps_body,
        grid=(2,),
        in_specs=[pl.BlockSpec((1, 3, N_), lambda i: (i, 0, 0))] * 3,
        out_specs=pl.BlockSpec((3, 1, 3, NPOINT_), lambda i: (0, i, 0, 0)),
        out_shape=jax.ShapeDtypeStruct((3, 2, 3, NPOINT_), jnp.float32),
        compiler_params=pltpu.CompilerParams(
            dimension_semantics=("parallel",)),
        interpret=interpret,
    )(xs, ys, zs)


def _topk_body(supp_ref, q_ref, o_ref):
    sT = supp_ref[0]
    q3 = q_ref[0]
    bt = pl.program_id(0)
    s2 = jnp.sum(sT * sT, axis=1, keepdims=True)
    qq = jnp.sum(q3 * q3, axis=0, keepdims=True)
    dot = jnp.dot(sT, q3, preferred_element_type=jnp.float32)
    d2 = (qq + s2) - 2.0 * dot
    siota = jax.lax.broadcasted_iota(jnp.int32, (N_, QB_), 0)
    rows = []
    am = None
    for _ in range(K_):
        if am is not None:
            d2 = jnp.where(siota == am, jnp.float32(jnp.inf), d2)
        v, idx = d2, siota
        size = N_ // 2
        while size >= 1:
            take = v[size:2 * size] < v[:size]
            v = jnp.where(take, v[size:2 * size], v[:size])
            idx = jnp.where(take, idx[size:2 * size], idx[:size])
            size //= 2
        am = idx
        rows.append(am)
    o_ref[0] = jnp.concatenate(rows, axis=0) + bt * N_


def _knn_topk(supp_t, points, interpret=False):
    TQ = points.shape[2]
    return pl.pallas_call(
        _topk_body,
        grid=(6, TQ // QB_),
        in_specs=[
            pl.BlockSpec((1, N_, 3), lambda bt, q: (bt, 0, 0)),
            pl.BlockSpec((1, 3, QB_), lambda bt, q: (bt // 3, 0, q)),
        ],
        out_specs=pl.BlockSpec((1, K_, QB_), lambda bt, q: (bt, 0, q)),
        out_shape=jax.ShapeDtypeStruct((6, K_, TQ), jnp.int32),
        compiler_params=pltpu.CompilerParams(
            dimension_semantics=("parallel", "parallel")),
        interpret=interpret,
    )(supp_t, points)


def _sc_gather(tab, nidx_flat):
    NI = nidx_flat.shape[0]
    W = 128
    idx2 = nidx_flat.reshape(1, NI)

    @pl.kernel(
        out_type=jax.ShapeDtypeStruct((NI, CPAD_), jnp.float32),
        mesh=plsc.VectorSubcoreMesh(core_axis_name="core",
                                    subcore_axis_name="subcore"),
    )
    def gat(tab_hbm, idx_hbm, o_hbm):
        def body(i_vmem, o_vmem):
            pltpu.sync_copy(tab_hbm.at[i_vmem.at[0]], o_vmem)

        pltpu.emit_pipeline(
            body,
            grid=(NI // W,),
            in_specs=[pl.BlockSpec((1, W), lambda i: (0, i))],
            out_specs=[pl.BlockSpec((W, CPAD_), lambda i: (i, 0))],
            core_axis_name=("core", "subcore"),
            dimension_semantics=(pltpu.PARALLEL,),
        )(idx_hbm, o_hbm)

    return gat(tab, idx2)


def _mlp_body(g_ref, q_ref, w0_ref, c0_ref, w1_ref, c1_ref, o_ref):
    X = g_ref[0].reshape(K_ * QB_, CPAD_)
    qT = q_ref[0]
    w0 = w0_ref[...]
    w1 = w1_ref[...]
    XW = jnp.dot(X, w0, preferred_element_type=jnp.float32)
    cq = c0_ref[...] - jnp.dot(qT, w0[0:3, :],
                               preferred_element_type=jnp.float32)
    h1 = XW.reshape(K_, QB_, 64) + cq[None, :, :]
    h1 = jnp.where(h1 >= 0, h1, 0.01 * h1)
    h2 = jnp.dot(h1.reshape(K_ * QB_, 64), w1,
                 preferred_element_type=jnp.float32) + c1_ref[...]
    h2 = jnp.where(h2 >= 0, h2, 0.01 * h2)
    r = jnp.max(h2.reshape(K_, QB_, 128), axis=0)
    t = pl.program_id(2)

    @pl.when(t == 0)
    def _():
        o_ref[0] = r

    @pl.when(t != 0)
    def _():
        o_ref[0] = jnp.maximum(o_ref[0], r)


def _mlp_max(gath, points_t, w0p, c0, w1p, c1, B, T, interpret=False):
    TQ = points_t.shape[1]
    NQ = TQ // QB_
    return pl.pallas_call(
        _mlp_body,
        grid=(B, NQ, T),
        in_specs=[
            pl.BlockSpec((1, K_, QB_, CPAD_),
                         lambda b, q, t: (b * T + t, 0, q, 0)),
            pl.BlockSpec((1, QB_, 3), lambda b, q, t: (b, q, 0)),
            pl.BlockSpec((CPAD_, 64), lambda b, q, t: (0, 0)),
            pl.BlockSpec((1, 64), lambda b, q, t: (0, 0)),
            pl.BlockSpec((64, 128), lambda b, q, t: (0, 0)),
            pl.BlockSpec((1, 128), lambda b, q, t: (0, 0)),
        ],
        out_specs=pl.BlockSpec((1, QB_, 128), lambda b, q, t: (b, q, 0)),
        out_shape=jax.ShapeDtypeStruct((B, TQ, 128), jnp.float32),
        compiler_params=pltpu.CompilerParams(
            dimension_semantics=("parallel", "parallel", "arbitrary")),
        interpret=interpret,
    )(gath, points_t, w0p, c0, w1p, c1)


def kernel(xyz, times, feat, W0, b0, g0, be0, m0, v0, W1, b1, g1, be1, m1, v1):
    B, T, _, N = xyz.shape
    C = feat.shape[1]
    TQ = T * NPOINT_

    s0 = g0 / jnp.sqrt(v0 + 1e-3)
    c0 = (b0 * s0 + be0 - m0 * s0).reshape(1, 64)
    W0p = W0 * s0[:, None]
    w0p = jnp.zeros((CPAD_, 64), jnp.float32).at[: 3 + C, :].set(W0p.T)
    s1 = g1 / jnp.sqrt(v1 + 1e-3)
    c1 = (b1 * s1 + be1 - m1 * s1).reshape(1, 128)
    w1p = (W1 * s1[:, None]).T

    xyz_flat = xyz.reshape(B * T, 3, N)

    psel = _fps_points(xyz_flat)
    points_t = jnp.transpose(psel, (1, 2, 3, 0)).reshape(B, TQ, 3)
    points = jnp.transpose(psel, (1, 0, 2, 3)).reshape(B, 3, TQ)
    supp_t = jnp.transpose(xyz_flat, (0, 2, 1))

    nidx = _knn_topk(supp_t, points)

    featT = jnp.transpose(feat, (0, 2, 1, 3))
    ptab = jnp.concatenate([xyz, featT], axis=2)
    ptab = jnp.transpose(ptab, (0, 1, 3, 2)).reshape(B * T * N, 3 + C)
    ptab = jnp.pad(ptab, ((0, 0), (0, CPAD_ - 3 - C)))
    gath = _sc_gather(ptab, nidx.reshape(-1))
    gath = gath.reshape(B * T, K_, TQ, CPAD_)

    out = _mlp_max(gath, points_t, w0p, c0, w1p, c1, B, T)
    return jnp.transpose(out, (0, 2, 1))


# speedup vs baseline: 1.4017x; 1.4017x over previous
import functools

import jax
import jax.numpy as jnp
from jax.experimental import pallas as pl
from jax.experimental.pallas import tpu as pltpu
from jax.experimental.pallas import tpu_sc as plsc

NPOINT_ = 1024
K_ = 32
N_ = 4096
QB_ = 1024
CPAD_ = 128


def _fps_body(xs_ref, ys_ref, zs_ref, o_ref):
    x = xs_ref[0]
    y = ys_ref[0]
    z = zs_ref[0]
    R = x.shape[0]
    lidx = jax.lax.broadcasted_iota(jnp.int32, (R, N_), 1)
    piota = jax.lax.broadcasted_iota(jnp.int32, (R, NPOINT_), 1)

    def step(i, carry):
        dist, far, ax, ay, az = carry
        sel = lidx == far
        cx = jnp.sum(jnp.where(sel, x, 0.0), axis=1, keepdims=True)
        cy = jnp.sum(jnp.where(sel, y, 0.0), axis=1, keepdims=True)
        cz = jnp.sum(jnp.where(sel, z, 0.0), axis=1, keepdims=True)
        hit = piota == i
        ax = jnp.where(hit, cx, ax)
        ay = jnp.where(hit, cy, ay)
        az = jnp.where(hit, cz, az)
        dx = x - cx
        dy = y - cy
        dz = z - cz
        d = (dx * dx + dy * dy) + dz * dz
        dist = jnp.minimum(dist, d)
        m = jnp.max(dist, axis=1, keepdims=True)
        far = jnp.min(jnp.where(dist >= m, lidx, jnp.int32(N_)), axis=1,
                      keepdims=True)
        return dist, far, ax, ay, az

    dist0 = jnp.full((R, N_), 1e10, jnp.float32)
    far0 = jnp.zeros((R, 1), jnp.int32)
    acc0 = jnp.zeros((R, NPOINT_), jnp.float32)
    _, _, ax, ay, az = jax.lax.fori_loop(0, NPOINT_, step,
                                         (dist0, far0, acc0, acc0, acc0))
    o_ref[0, 0] = ax
    o_ref[1, 0] = ay
    o_ref[2, 0] = az


def _fps_points(xyz_flat, interpret=False):
    xs = xyz_flat[:, 0, :].reshape(2, 3, N_)
    ys = xyz_flat[:, 1, :].reshape(2, 3, N_)
    zs = xyz_flat[:, 2, :].reshape(2, 3, N_)
    return pl.pallas_call(
        _fps_body,
        grid=(2,),
        in_specs=[pl.BlockSpec((1, 3, N_), lambda i: (i, 0, 0))] * 3,
        out_specs=pl.BlockSpec((3, 1, 3, NPOINT_), lambda i: (0, i, 0, 0)),
        out_shape=jax.ShapeDtypeStruct((3, 2, 3, NPOINT_), jnp.float32),
        compiler_params=pltpu.CompilerParams(
            dimension_semantics=("parallel",)),
        interpret=interpret,
    )(xs, ys, zs)


def _topk_body(supp_ref, q_ref, o_ref):
    s = supp_ref[0]
    qT = q_ref[0]
    bt = pl.program_id(0)
    s2 = jnp.sum(s * s, axis=0, keepdims=True)
    qq = jnp.sum(qT * qT, axis=1, keepdims=True)
    dot = jnp.dot(qT, s, preferred_element_type=jnp.float32)
    d2 = (qq + s2) - 2.0 * dot
    lidx = jax.lax.broadcasted_iota(jnp.int32, (QB_, N_), 1)
    cols = []
    for _ in range(K_):
        m = jnp.min(d2, axis=1, keepdims=True)
        am = jnp.min(jnp.where(d2 <= m, lidx, jnp.int32(N_)), axis=1,
                     keepdims=True)
        cols.append(am)
        d2 = jnp.where(lidx == am, jnp.float32(jnp.inf), d2)
    o_ref[0] = jnp.concatenate(cols, axis=1) + bt * N_


def _knn_topk(xyz_flat, points_t, interpret=False):
    TQ = points_t.shape[1]
    return pl.pallas_call(
        _topk_body,
        grid=(6, TQ // QB_),
        in_specs=[
            pl.BlockSpec((1, 3, N_), lambda bt, q: (bt, 0, 0)),
            pl.BlockSpec((1, QB_, 3), lambda bt, q: (bt // 3, q, 0)),
        ],
        out_specs=pl.BlockSpec((1, QB_, K_), lambda bt, q: (bt, q, 0)),
        out_shape=jax.ShapeDtypeStruct((6, TQ, K_), jnp.int32),
        compiler_params=pltpu.CompilerParams(
            dimension_semantics=("parallel", "parallel")),
        interpret=interpret,
    )(xyz_flat, points_t)


def _sc_gather(tab, nidx_flat):
    NI = nidx_flat.shape[0]
    W = 128
    idx2 = nidx_flat.reshape(1, NI)

    @pl.kernel(
        out_type=jax.ShapeDtypeStruct((NI, CPAD_), jnp.float32),
        mesh=plsc.VectorSubcoreMesh(core_axis_name="core",
                                    subcore_axis_name="subcore"),
    )
    def gat(tab_hbm, idx_hbm, o_hbm):
        def body(i_vmem, o_vmem):
            pltpu.sync_copy(tab_hbm.at[i_vmem.at[0]], o_vmem)

        pltpu.emit_pipeline(
            body,
            grid=(NI // W,),
            in_specs=[pl.BlockSpec((1, W), lambda i: (0, i))],
            out_specs=[pl.BlockSpec((W, CPAD_), lambda i: (i, 0))],
            core_axis_name=("core", "subcore"),
            dimension_semantics=(pltpu.PARALLEL,),
        )(idx_hbm, o_hbm)

    return gat(tab, idx2)


def _mlp_body(g_ref, q_ref, w0_ref, c0_ref, w1_ref, c1_ref, o_ref):
    X = g_ref[0]
    qT = q_ref[0]
    w0 = w0_ref[...]
    w1 = w1_ref[...]
    XW = jnp.dot(X, w0, preferred_element_type=jnp.float32)
    cq = c0_ref[...] - jnp.dot(qT, w0[0:3, :],
                               preferred_element_type=jnp.float32)
    h1 = XW.reshape(QB_, K_, 64) + cq[:, None, :]
    h1 = jnp.where(h1 >= 0, h1, 0.01 * h1)
    h2 = jnp.dot(h1.reshape(QB_ * K_, 64), w1,
                 preferred_element_type=jnp.float32) + c1_ref[...]
    h2 = jnp.where(h2 >= 0, h2, 0.01 * h2)
    r = jnp.max(h2.reshape(QB_, K_, 128), axis=1)
    t = pl.program_id(2)

    @pl.when(t == 0)
    def _():
        o_ref[0] = r

    @pl.when(t != 0)
    def _():
        o_ref[0] = jnp.maximum(o_ref[0], r)


def _mlp_max(gath, points_t, w0p, c0, w1p, c1, B, T, interpret=False):
    TQ = points_t.shape[1]
    NQ = TQ // QB_
    return pl.pallas_call(
        _mlp_body,
        grid=(B, NQ, T),
        in_specs=[
            pl.BlockSpec((1, QB_ * K_, CPAD_),
                         lambda b, q, t: (b * T + t, q, 0)),
            pl.BlockSpec((1, QB_, 3), lambda b, q, t: (b, q, 0)),
            pl.BlockSpec((CPAD_, 64), lambda b, q, t: (0, 0)),
            pl.BlockSpec((1, 64), lambda b, q, t: (0, 0)),
            pl.BlockSpec((64, 128), lambda b, q, t: (0, 0)),
            pl.BlockSpec((1, 128), lambda b, q, t: (0, 0)),
        ],
        out_specs=pl.BlockSpec((1, QB_, 128), lambda b, q, t: (b, q, 0)),
        out_shape=jax.ShapeDtypeStruct((B, TQ, 128), jnp.float32),
        compiler_params=pltpu.CompilerParams(
            dimension_semantics=("parallel", "parallel", "arbitrary")),
        interpret=interpret,
    )(gath, points_t, w0p, c0, w1p, c1)


def kernel(xyz, times, feat, W0, b0, g0, be0, m0, v0, W1, b1, g1, be1, m1, v1):
    B, T, _, N = xyz.shape
    C = feat.shape[1]
    TQ = T * NPOINT_

    s0 = g0 / jnp.sqrt(v0 + 1e-3)
    c0 = (b0 * s0 + be0 - m0 * s0).reshape(1, 64)
    W0p = W0 * s0[:, None]
    w0p = jnp.zeros((CPAD_, 64), jnp.float32).at[: 3 + C, :].set(W0p.T)
    s1 = g1 / jnp.sqrt(v1 + 1e-3)
    c1 = (b1 * s1 + be1 - m1 * s1).reshape(1, 128)
    w1p = (W1 * s1[:, None]).T

    xyz_flat = xyz.reshape(B * T, 3, N)

    psel = _fps_points(xyz_flat)
    points_t = jnp.transpose(psel, (1, 2, 3, 0)).reshape(B, TQ, 3)

    nidx = _knn_topk(xyz_flat, points_t)

    featT = jnp.transpose(feat, (0, 2, 1, 3))
    ptab = jnp.concatenate([xyz, featT], axis=2)
    ptab = jnp.transpose(ptab, (0, 1, 3, 2)).reshape(B * T * N, 3 + C)
    ptab = jnp.pad(ptab, ((0, 0), (0, CPAD_ - 3 - C)))
    gath = _sc_gather(ptab, nidx.reshape(-1))
    gath = gath.reshape(B * T, TQ * K_, CPAD_)

    out = _mlp_max(gath, points_t, w0p, c0, w1p, c1, B, T)
    return jnp.transpose(out, (0, 2, 1))


# speedup vs baseline: 1.5741x; 1.1230x over previous
import functools

import jax
import jax.numpy as jnp
from jax.experimental import pallas as pl
from jax.experimental.pallas import tpu as pltpu
from jax.experimental.pallas import tpu_sc as plsc

NPOINT_ = 1024
K_ = 32
N_ = 4096
QB_ = 1024
CPAD_ = 128


def _fps_body(xs_ref, ys_ref, zs_ref, o_ref):
    x = xs_ref[0]
    y = ys_ref[0]
    z = zs_ref[0]
    R = x.shape[0]
    lidx = jax.lax.broadcasted_iota(jnp.int32, (R, N_), 1)
    piota = jax.lax.broadcasted_iota(jnp.int32, (R, NPOINT_), 1)

    def step(i, carry):
        dist, far, ax, ay, az = carry
        sel = lidx == far
        cx = jnp.sum(jnp.where(sel, x, 0.0), axis=1, keepdims=True)
        cy = jnp.sum(jnp.where(sel, y, 0.0), axis=1, keepdims=True)
        cz = jnp.sum(jnp.where(sel, z, 0.0), axis=1, keepdims=True)
        hit = piota == i
        ax = jnp.where(hit, cx, ax)
        ay = jnp.where(hit, cy, ay)
        az = jnp.where(hit, cz, az)
        dx = x - cx
        dy = y - cy
        dz = z - cz
        d = (dx * dx + dy * dy) + dz * dz
        dist = jnp.minimum(dist, d)
        m = jnp.max(dist, axis=1, keepdims=True)
        far = jnp.min(jnp.where(dist >= m, lidx, jnp.int32(N_)), axis=1,
                      keepdims=True)
        return dist, far, ax, ay, az

    dist0 = jnp.full((R, N_), 1e10, jnp.float32)
    far0 = jnp.zeros((R, 1), jnp.int32)
    acc0 = jnp.zeros((R, NPOINT_), jnp.float32)
    _, _, ax, ay, az = jax.lax.fori_loop(0, NPOINT_, step,
                                         (dist0, far0, acc0, acc0, acc0))
    o_ref[0, 0] = ax
    o_ref[1, 0] = ay
    o_ref[2, 0] = az


def _fps_points(xyz_flat, interpret=False):
    xs = xyz_flat[:, 0, :].reshape(1, 6, N_)
    ys = xyz_flat[:, 1, :].reshape(1, 6, N_)
    zs = xyz_flat[:, 2, :].reshape(1, 6, N_)
    out = pl.pallas_call(
        _fps_body,
        grid=(1,),
        in_specs=[pl.BlockSpec((1, 6, N_), lambda i: (0, 0, 0))] * 3,
        out_specs=pl.BlockSpec((3, 1, 6, NPOINT_), lambda i: (0, 0, 0, 0)),
        out_shape=jax.ShapeDtypeStruct((3, 1, 6, NPOINT_), jnp.float32),
        compiler_params=pltpu.CompilerParams(
            dimension_semantics=("arbitrary",)),
        interpret=interpret,
    )(xs, ys, zs)
    return out.reshape(3, 2, 3, NPOINT_)


def _topk_body(supp_ref, q_ref, o_ref):
    s = supp_ref[0]
    qT = q_ref[0]
    bt = pl.program_id(0)
    s2 = jnp.sum(s * s, axis=0, keepdims=True)
    qq = jnp.sum(qT * qT, axis=1, keepdims=True)
    dot = jnp.dot(qT, s, preferred_element_type=jnp.float32)
    d2 = (qq + s2) - 2.0 * dot
    lidx = jax.lax.broadcasted_iota(jnp.int32, (QB_, N_), 1)
    cols = []
    for _ in range(K_):
        m = jnp.min(d2, axis=1, keepdims=True)
        am = jnp.min(jnp.where(d2 <= m, lidx, jnp.int32(N_)), axis=1,
                     keepdims=True)
        cols.append(am)
        d2 = jnp.where(lidx == am, jnp.float32(jnp.inf), d2)
    o_ref[0] = jnp.concatenate(cols, axis=1) + bt * N_


def _knn_topk(xyz_flat, points_t, interpret=False):
    TQ = points_t.shape[1]
    return pl.pallas_call(
        _topk_body,
        grid=(6, TQ // QB_),
        in_specs=[
            pl.BlockSpec((1, 3, N_), lambda bt, q: (bt, 0, 0)),
            pl.BlockSpec((1, QB_, 3), lambda bt, q: (bt // 3, q, 0)),
        ],
        out_specs=pl.BlockSpec((1, QB_, K_), lambda bt, q: (bt, q, 0)),
        out_shape=jax.ShapeDtypeStruct((6, TQ, K_), jnp.int32),
        compiler_params=pltpu.CompilerParams(
            dimension_semantics=("parallel", "parallel")),
        interpret=interpret,
    )(xyz_flat, points_t)


def _sc_gather(tab, nidx_flat):
    NI = nidx_flat.shape[0]
    W = 128
    idx2 = nidx_flat.reshape(1, NI)

    @pl.kernel(
        out_type=jax.ShapeDtypeStruct((NI, CPAD_), jnp.float32),
        mesh=plsc.VectorSubcoreMesh(core_axis_name="core",
                                    subcore_axis_name="subcore"),
    )
    def gat(tab_hbm, idx_hbm, o_hbm):
        def body(i_vmem, o_vmem):
            pltpu.sync_copy(tab_hbm.at[i_vmem.at[0]], o_vmem)

        pltpu.emit_pipeline(
            body,
            grid=(NI // W,),
            in_specs=[pl.BlockSpec((1, W), lambda i: (0, i))],
            out_specs=[pl.BlockSpec((W, CPAD_), lambda i: (i, 0))],
            core_axis_name=("core", "subcore"),
            dimension_semantics=(pltpu.PARALLEL,),
        )(idx_hbm, o_hbm)

    return gat(tab, idx2)


def _mlp_body(g_ref, q_ref, w0_ref, c0_ref, w1_ref, c1_ref, o_ref):
    X = g_ref[0]
    qT = q_ref[0]
    w0 = w0_ref[...]
    w1 = w1_ref[...]
    XW = jnp.dot(X, w0, preferred_element_type=jnp.float32)
    cq = c0_ref[...] - jnp.dot(qT, w0[0:3, :],
                               preferred_element_type=jnp.float32)
    h1 = XW.reshape(QB_, K_, 64) + cq[:, None, :]
    h1 = jnp.where(h1 >= 0, h1, 0.01 * h1)
    h2 = jnp.dot(h1.reshape(QB_ * K_, 64), w1,
                 preferred_element_type=jnp.float32) + c1_ref[...]
    h2 = jnp.where(h2 >= 0, h2, 0.01 * h2)
    r = jnp.max(h2.reshape(QB_, K_, 128), axis=1)
    t = pl.program_id(2)

    @pl.when(t == 0)
    def _():
        o_ref[0] = r

    @pl.when(t != 0)
    def _():
        o_ref[0] = jnp.maximum(o_ref[0], r)


def _mlp_max(gath, points_t, w0p, c0, w1p, c1, B, T, interpret=False):
    TQ = points_t.shape[1]
    NQ = TQ // QB_
    return pl.pallas_call(
        _mlp_body,
        grid=(B, NQ, T),
        in_specs=[
            pl.BlockSpec((1, QB_ * K_, CPAD_),
                         lambda b, q, t: (b * T + t, q, 0)),
            pl.BlockSpec((1, QB_, 3), lambda b, q, t: (b, q, 0)),
            pl.BlockSpec((CPAD_, 64), lambda b, q, t: (0, 0)),
            pl.BlockSpec((1, 64), lambda b, q, t: (0, 0)),
            pl.BlockSpec((64, 128), lambda b, q, t: (0, 0)),
            pl.BlockSpec((1, 128), lambda b, q, t: (0, 0)),
        ],
        out_specs=pl.BlockSpec((1, QB_, 128), lambda b, q, t: (b, q, 0)),
        out_shape=jax.ShapeDtypeStruct((B, TQ, 128), jnp.float32),
        compiler_params=pltpu.CompilerParams(
            dimension_semantics=("parallel", "parallel", "arbitrary")),
        interpret=interpret,
    )(gath, points_t, w0p, c0, w1p, c1)


def kernel(xyz, times, feat, W0, b0, g0, be0, m0, v0, W1, b1, g1, be1, m1, v1):
    B, T, _, N = xyz.shape
    C = feat.shape[1]
    TQ = T * NPOINT_

    s0 = g0 / jnp.sqrt(v0 + 1e-3)
    c0 = (b0 * s0 + be0 - m0 * s0).reshape(1, 64)
    W0p = W0 * s0[:, None]
    w0p = jnp.zeros((CPAD_, 64), jnp.float32).at[: 3 + C, :].set(W0p.T)
    s1 = g1 / jnp.sqrt(v1 + 1e-3)
    c1 = (b1 * s1 + be1 - m1 * s1).reshape(1, 128)
    w1p = (W1 * s1[:, None]).T

    xyz_flat = xyz.reshape(B * T, 3, N)

    psel = _fps_points(xyz_flat)
    points_t = jnp.transpose(psel, (1, 2, 3, 0)).reshape(B, TQ, 3)

    nidx = _knn_topk(xyz_flat, points_t)

    featT = jnp.transpose(feat, (0, 2, 1, 3))
    ptab = jnp.concatenate([xyz, featT], axis=2)
    ptab = jnp.transpose(ptab, (0, 1, 3, 2)).reshape(B * T * N, 3 + C)
    ptab = jnp.pad(ptab, ((0, 0), (0, CPAD_ - 3 - C)))
    gath = _sc_gather(ptab, nidx.reshape(-1))
    gath = gath.reshape(B * T, TQ * K_, CPAD_)

    out = _mlp_max(gath, points_t, w0p, c0, w1p, c1, B, T)
    return jnp.transpose(out, (0, 2, 1))
